# Initial kernel scaffold; baseline (speedup 1.0000x reference)
import jax
import jax.numpy as jnp
from jax.experimental import pallas as pl


def kernel(w1, b1, w2, b2, x):
    raise NotImplementedError("write your pallas kernel here")


# speedup vs baseline: 1.0495x; 1.0495x over previous
import functools

import jax
import jax.numpy as jnp
from jax.experimental import pallas as pl
from jax.experimental.pallas import tpu as pltpu


def _se_kernel(x_ref, w1_ref, b1_ref, w2_ref, b2_ref, o_ref):
    x = x_ref[...]
    s = jnp.sum(x, axis=2)
    h = jnp.maximum(
        jnp.dot(s, w1_ref[...], preferred_element_type=jnp.float32)
        + b1_ref[...], 0.0)
    g = jnp.clip(
        jnp.dot(h, w2_ref[...], preferred_element_type=jnp.float32)
        + b2_ref[...], 0.0, 1.0)
    o_ref[...] = x * g[:, :, None]


def _largest_divisor(n, cap):
    d = min(n, cap)
    while n % d:
        d -= 1
    return d


def kernel(w1, b1, w2, b2, x):
    N, C, H, W = x.shape
    HW = H * W
    x3 = x.reshape(N, C, HW)

    inv_hw = 1.0 / float(HW)
    w1s = w1 * inv_hw
    w2s = w2 * (1.0 / 6.0)
    b2s = b2 * (1.0 / 6.0) + 0.5

    slab = C * HW * 4
    bn = _largest_divisor(N, max(1, (6 << 20) // slab))
    if N // bn < 2 and N >= 2:
        bn = _largest_divisor(N, max(1, bn // 2))
    grid = (N // bn,)

    block_bytes = bn * slab
    vmem_limit = min(4 * block_bytes + (8 << 20), 60 << 20)

    out = pl.pallas_call(
        _se_kernel,
        out_shape=jax.ShapeDtypeStruct((N, C, HW), x.dtype),
        grid=grid,
        in_specs=[
            pl.BlockSpec((bn, C, HW), lambda i: (i, 0, 0)),
            pl.BlockSpec(w1s.shape, lambda i: (0, 0)),
            pl.BlockSpec(b1.shape, lambda i: (0, 0)),
            pl.BlockSpec(w2s.shape, lambda i: (0, 0)),
            pl.BlockSpec(b2s.shape, lambda i: (0, 0)),
        ],
        out_specs=pl.BlockSpec((bn, C, HW), lambda i: (i, 0, 0)),
        compiler_params=pltpu.CompilerParams(
            dimension_semantics=("parallel",),
            vmem_limit_bytes=vmem_limit,
        ),
    )(x3, w1s, b1, w2s, b2s)
    return out.reshape(N, C, H, W)


# speedup vs baseline: 4.5905x; 4.3740x over previous
import jax
import jax.numpy as jnp
from jax.experimental import pallas as pl
from jax.experimental.pallas import tpu as pltpu


def _se_kernel(x_ref, w1_ref, b1_ref, w2_ref, b2_ref, o_ref):
    x = x_ref[...]
    s = jnp.sum(x, axis=0)
    h = jnp.maximum(
        jnp.dot(s, w1_ref[...], preferred_element_type=jnp.float32)
        + b1_ref[...], 0.0)
    g = jnp.clip(
        jnp.dot(h, w2_ref[...], preferred_element_type=jnp.float32)
        + b2_ref[...], 0.0, 1.0)
    o_ref[...] = x * g[None, :, :]


def _largest_divisor(n, cap, align=1):
    cap = max(align, min(n, cap))
    d = (cap // align) * align
    while d >= align:
        if n % d == 0:
            return d
        d -= align
    return n


def kernel(w1, b1, w2, b2, x):
    N, C, H, W = x.shape
    HW = H * W

    xt = x.transpose(2, 3, 0, 1).reshape(HW, N, C)

    w1s = w1 * (1.0 / float(HW))
    w2s = w2 * (1.0 / 6.0)
    b2s = b2 * (1.0 / 6.0) + 0.5

    slab = HW * C * 4
    bn = _largest_divisor(N, max(1, (6 << 20) // slab), align=8 if N % 8 == 0 else 1)
    if N // bn < 2 and N >= 2:
        bn = _largest_divisor(N, max(1, bn // 2))
    grid = (N // bn,)

    block_bytes = bn * slab
    vmem_limit = min(4 * block_bytes + (8 << 20), 60 << 20)

    out = pl.pallas_call(
        _se_kernel,
        out_shape=jax.ShapeDtypeStruct((HW, N, C), x.dtype),
        grid=grid,
        in_specs=[
            pl.BlockSpec((HW, bn, C), lambda i: (0, i, 0)),
            pl.BlockSpec(w1s.shape, lambda i: (0, 0)),
            pl.BlockSpec(b1.shape, lambda i: (0, 0)),
            pl.BlockSpec(w2s.shape, lambda i: (0, 0)),
            pl.BlockSpec(b2s.shape, lambda i: (0, 0)),
        ],
        out_specs=pl.BlockSpec((HW, bn, C), lambda i: (0, i, 0)),
        compiler_params=pltpu.CompilerParams(
            dimension_semantics=("parallel",),
            vmem_limit_bytes=vmem_limit,
        ),
    )(xt, w1s, b1, w2s, b2s)

    return out.reshape(H, W, N, C).transpose(2, 3, 0, 1)


# speedup vs baseline: 4.5960x; 1.0012x over previous
import jax
import jax.numpy as jnp
from jax.experimental import pallas as pl
from jax.experimental.pallas import tpu as pltpu


def _se_kernel(x_ref, w1_ref, b1_ref, w2_ref, b2_ref, o_ref):
    x = x_ref[...]
    s = jnp.sum(x, axis=0)
    h = jnp.maximum(
        jnp.dot(s, w1_ref[...], preferred_element_type=jnp.float32)
        + b1_ref[...], 0.0)
    g = jnp.clip(
        jnp.dot(h, w2_ref[...], preferred_element_type=jnp.float32)
        + b2_ref[...], 0.0, 1.0)
    o_ref[...] = x * g[None, :, :]


def _largest_divisor(n, cap, align=1):
    cap = max(align, min(n, cap))
    d = (cap // align) * align
    while d >= align:
        if n % d == 0:
            return d
        d -= align
    return n


def kernel(w1, b1, w2, b2, x):
    N, C, H, W = x.shape
    HW = H * W

    xt = x.transpose(2, 3, 0, 1).reshape(HW, N, C)

    w1s = w1 * (1.0 / float(HW))
    w2s = w2 * (1.0 / 6.0)
    b2s = b2 * (1.0 / 6.0) + 0.5

    slab = HW * C * 4
    bn = _largest_divisor(N, max(1, (3 << 20) // slab), align=8 if N % 8 == 0 else 1)
    if N // bn < 2 and N >= 2:
        bn = _largest_divisor(N, max(1, bn // 2))
    grid = (N // bn,)

    block_bytes = bn * slab
    vmem_limit = min(4 * block_bytes + (8 << 20), 60 << 20)

    out = pl.pallas_call(
        _se_kernel,
        out_shape=jax.ShapeDtypeStruct((HW, N, C), x.dtype),
        grid=grid,
        in_specs=[
            pl.BlockSpec((HW, bn, C), lambda i: (0, i, 0)),
            pl.BlockSpec(w1s.shape, lambda i: (0, 0)),
            pl.BlockSpec(b1.shape, lambda i: (0, 0)),
            pl.BlockSpec(w2s.shape, lambda i: (0, 0)),
            pl.BlockSpec(b2s.shape, lambda i: (0, 0)),
        ],
        out_specs=pl.BlockSpec((HW, bn, C), lambda i: (0, i, 0)),
        compiler_params=pltpu.CompilerParams(
            dimension_semantics=("parallel",),
            vmem_limit_bytes=vmem_limit,
        ),
    )(xt, w1s, b1, w2s, b2s)

    return out.reshape(H, W, N, C).transpose(2, 3, 0, 1)


# speedup vs baseline: 4.5987x; 1.0006x over previous
import functools

import jax
import jax.numpy as jnp
from jax.experimental import pallas as pl
from jax.experimental.pallas import tpu as pltpu


def _se_kernel(x_ref, w1_ref, b1_ref, w2_ref, b2_ref, o_ref, *, inv_hw):
    x = x_ref[...]
    s = jnp.sum(x, axis=0) * inv_hw
    h = jnp.maximum(
        jnp.dot(s, w1_ref[...], preferred_element_type=jnp.float32)
        + b1_ref[...], 0.0)
    z = (jnp.dot(h, w2_ref[...], preferred_element_type=jnp.float32)
         + b2_ref[...]) * (1.0 / 6.0) + 0.5
    g = jnp.clip(z, 0.0, 1.0)
    o_ref[...] = x * g[None, :, :]


def _largest_divisor(n, cap, align=1):
    cap = max(align, min(n, cap))
    d = (cap // align) * align
    while d >= align:
        if n % d == 0:
            return d
        d -= align
    return n


def kernel(w1, b1, w2, b2, x):
    N, C, H, W = x.shape
    HW = H * W

    xt = x.transpose(2, 3, 0, 1).reshape(HW, N, C)

    slab = HW * C * 4
    bn = _largest_divisor(N, max(1, (3 << 20) // slab), align=8 if N % 8 == 0 else 1)
    if N // bn < 2 and N >= 2:
        bn = _largest_divisor(N, max(1, bn // 2))
    grid = (N // bn,)

    block_bytes = bn * slab
    vmem_limit = min(4 * block_bytes + (8 << 20), 60 << 20)

    out = pl.pallas_call(
        functools.partial(_se_kernel, inv_hw=1.0 / float(HW)),
        out_shape=jax.ShapeDtypeStruct((HW, N, C), x.dtype),
        grid=grid,
        in_specs=[
            pl.BlockSpec((HW, bn, C), lambda i: (0, i, 0)),
            pl.BlockSpec(w1.shape, lambda i: (0, 0)),
            pl.BlockSpec(b1.shape, lambda i: (0, 0)),
            pl.BlockSpec(w2.shape, lambda i: (0, 0)),
            pl.BlockSpec(b2.shape, lambda i: (0, 0)),
        ],
        out_specs=pl.BlockSpec((HW, bn, C), lambda i: (0, i, 0)),
        compiler_params=pltpu.CompilerParams(
            dimension_semantics=("parallel",),
            vmem_limit_bytes=vmem_limit,
        ),
    )(xt, w1, b1, w2, b2)

    return out.reshape(H, W, N, C).transpose(2, 3, 0, 1)


# speedup vs baseline: 4.7682x; 1.0368x over previous
import functools

import jax
import jax.numpy as jnp
from jax.experimental import pallas as pl
from jax.experimental.pallas import tpu as pltpu


def _se_kernel(x_ref, p1_ref, p2_ref, o_ref, *, inv_hw, c_in, c_mid):
    x = x_ref[...]
    w1 = p1_ref[:c_in, :]
    b1 = p1_ref[c_in:c_in + 1, :]
    w2 = p2_ref[:c_mid, :]
    b2 = p2_ref[c_mid:c_mid + 1, :]
    s = jnp.sum(x, axis=0) * inv_hw
    h = jnp.maximum(
        jnp.dot(s, w1, preferred_element_type=jnp.float32) + b1, 0.0)
    z = (jnp.dot(h, w2, preferred_element_type=jnp.float32) + b2) \
        * (1.0 / 6.0) + 0.5
    g = jnp.clip(z, 0.0, 1.0)
    o_ref[...] = x * g[None, :, :]


def _largest_divisor(n, cap, align=1):
    cap = max(align, min(n, cap))
    d = (cap // align) * align
    while d >= align:
        if n % d == 0:
            return d
        d -= align
    return n


def kernel(w1, b1, w2, b2, x):
    N, C, H, W = x.shape
    HW = H * W

    xt = x.transpose(2, 3, 0, 1).reshape(HW, N, C)

    Cr = w1.shape[1]
    p1 = jnp.concatenate([w1, b1], axis=0)
    p2 = jnp.concatenate([w2, b2], axis=0)

    slab = HW * C * 4
    bn = _largest_divisor(N, max(1, (3 << 20) // slab), align=8 if N % 8 == 0 else 1)
    if N // bn < 2 and N >= 2:
        bn = _largest_divisor(N, max(1, bn // 2))
    grid = (N // bn,)

    block_bytes = bn * slab
    vmem_limit = min(4 * block_bytes + (8 << 20), 60 << 20)

    out = pl.pallas_call(
        functools.partial(_se_kernel, inv_hw=1.0 / float(HW),
                          c_in=C, c_mid=Cr),
        out_shape=jax.ShapeDtypeStruct((HW, N, C), x.dtype),
        grid=grid,
        in_specs=[
            pl.BlockSpec((HW, bn, C), lambda i: (0, i, 0)),
            pl.BlockSpec(p1.shape, lambda i: (0, 0)),
            pl.BlockSpec(p2.shape, lambda i: (0, 0)),
        ],
        out_specs=pl.BlockSpec((HW, bn, C), lambda i: (0, i, 0)),
        compiler_params=pltpu.CompilerParams(
            dimension_semantics=("parallel",),
            vmem_limit_bytes=vmem_limit,
        ),
    )(xt, p1, p2)

    return out.reshape(H, W, N, C).transpose(2, 3, 0, 1)


# speedup vs baseline: 4.8562x; 1.0185x over previous
import functools

import jax
import jax.numpy as jnp
from jax.experimental import pallas as pl
from jax.experimental.pallas import tpu as pltpu


def _se_kernel(x_ref, p_ref, o_ref, *, inv_hw, c_in, c_mid):
    x = x_ref[...]
    r2 = c_in + 8
    w1 = p_ref[:c_in, :c_mid]
    b1 = p_ref[c_in:c_in + 1, :c_mid]
    w2 = p_ref[r2:r2 + c_mid, :]
    b2 = p_ref[r2 + c_mid:r2 + c_mid + 1, :]
    s = jnp.sum(x, axis=0) * inv_hw
    h = jnp.maximum(
        jnp.dot(s, w1, preferred_element_type=jnp.float32) + b1, 0.0)
    z = (jnp.dot(h, w2, preferred_element_type=jnp.float32) + b2) \
        * (1.0 / 6.0) + 0.5
    g = jnp.clip(z, 0.0, 1.0)
    o_ref[...] = x * g[None, :, :]


def _largest_divisor(n, cap, align=1):
    cap = max(align, min(n, cap))
    d = (cap // align) * align
    while d >= align:
        if n % d == 0:
            return d
        d -= align
    return n


def kernel(w1, b1, w2, b2, x):
    N, C, H, W = x.shape
    HW = H * W

    xt = x.transpose(2, 3, 0, 1).reshape(HW, N, C)

    Cr = w1.shape[1]
    p = jnp.concatenate([
        jnp.pad(jnp.concatenate([w1, b1], axis=0),
                ((0, 7), (0, C - Cr))),
        w2, b2], axis=0)

    slab = HW * C * 4
    bn = _largest_divisor(N, max(1, (3 << 20) // slab), align=8 if N % 8 == 0 else 1)
    if N // bn < 2 and N >= 2:
        bn = _largest_divisor(N, max(1, bn // 2))
    grid = (N // bn,)

    block_bytes = bn * slab
    vmem_limit = min(4 * block_bytes + (8 << 20), 60 << 20)

    out = pl.pallas_call(
        functools.partial(_se_kernel, inv_hw=1.0 / float(HW),
                          c_in=C, c_mid=Cr),
        out_shape=jax.ShapeDtypeStruct((HW, N, C), x.dtype),
        grid=grid,
        in_specs=[
            pl.BlockSpec((HW, bn, C), lambda i: (0, i, 0)),
            pl.BlockSpec(p.shape, lambda i: (0, 0)),
        ],
        out_specs=pl.BlockSpec((HW, bn, C), lambda i: (0, i, 0)),
        compiler_params=pltpu.CompilerParams(
            dimension_semantics=("parallel",),
            vmem_limit_bytes=vmem_limit,
        ),
    )(xt, p)

    return out.reshape(H, W, N, C).transpose(2, 3, 0, 1)
